# baseline (speedup 1.0000x reference)
import functools

import jax
import jax.numpy as jnp
from jax import lax
from jax.experimental import pallas as pl
from jax.experimental.pallas import tpu as pltpu
from jax.experimental.pallas import tpu_sc as plsc

N = 10000
E = 320000
D = 128
G = 32

NP = 10240
NTILES = 32
EPT = E // NTILES
CH = 125
NCH = EPT // CH
ROWS_PT = NP // 16

BLK = 2048
GRID = NP // BLK

f32 = jnp.float32
bf16 = jnp.bfloat16

_mesh = plsc.VectorSubcoreMesh(core_axis_name="c", subcore_axis_name="s")


def _sc_mp_body(g_hbm, edges_hbm, zero_hbm, acc_hbm,
                edge_buf, buf0, sem0, acc_sh):
    c = lax.axis_index("c")
    s = lax.axis_index("s")
    wid = c * 16 + s
    pltpu.sync_copy(edges_hbm.at[wid], edge_buf)
    src_buf = edge_buf.at[0]
    dst_buf = edge_buf.at[1]
    rows = pl.ds(s * ROWS_PT, ROWS_PT)
    pltpu.sync_copy(zero_hbm.at[rows], acc_sh.at[rows])
    plsc.subcore_barrier()

    def body(j, carry):
        pltpu.async_copy(g_hbm.at[src_buf.at[j]], buf0, sem0).wait()
        pltpu.sync_copy(buf0, acc_sh.at[dst_buf.at[j]], add=True)
        return carry

    lax.fori_loop(0, NCH, body, 0)
    plsc.subcore_barrier()
    pltpu.sync_copy(acc_sh.at[rows], acc_hbm.at[c].at[rows])


_sc_mp = pl.kernel(
    _sc_mp_body,
    out_type=jax.ShapeDtypeStruct((2, NP, D), f32),
    mesh=_mesh,
    scratch_types=[
        pltpu.VMEM((2, NCH, CH), jnp.int32),
        pltpu.VMEM((CH, D), f32),
        pltpu.SemaphoreType.DMA,
        pltpu.VMEM_SHARED((NP, D), f32),
    ],
)

def _tc0_body(dacc_ref, x_ref, w_ref, bin_ref, W_ref, g_ref, dis_ref):
    deg = dacc_ref[0][:, 0:1] + dacc_ref[1][:, 0:1] + 1.0
    dis = 1.0 / jnp.sqrt(deg)
    h0 = x_ref[...] * w_ref[...] + bin_ref[...]
    t = jnp.dot(h0.astype(bf16), W_ref[...].astype(bf16),
                preferred_element_type=f32)
    g_ref[...] = dis * t
    dis_ref[...] = dis


def _tc_mid_body(acc_ref, gprev_ref, dis_ref, b_ref, W_ref, g_ref):
    dis = dis_ref[...]
    h = dis * (acc_ref[0] + acc_ref[1] + gprev_ref[...]) + b_ref[...]
    t = jnp.dot(h.astype(bf16), W_ref[...].astype(bf16),
                preferred_element_type=f32)
    g_ref[...] = dis * t


def _tc_fin_body(acc_ref, gprev_ref, dis_ref, bg2_ref, batch_ref,
                 Wr1_ref, br1_ref, Wr2_ref, br2_ref, out_ref, pooled_ref):
    i = pl.program_id(0)
    h3 = (dis_ref[...] * (acc_ref[0] + acc_ref[1] + gprev_ref[...])
          + bg2_ref[...])
    bb = batch_ref[...][:, 0]
    gids = lax.broadcasted_iota(jnp.int32, (G, BLK), 0)
    Bt = (bb[None, :] == gids).astype(bf16)
    hi = h3.astype(bf16)
    r = h3 - hi.astype(f32)
    mid = r.astype(bf16)
    lo = (r - mid.astype(f32)).astype(bf16)
    part = (jnp.dot(Bt, hi, preferred_element_type=f32)
            + jnp.dot(Bt, mid, preferred_element_type=f32)
            + jnp.dot(Bt, lo, preferred_element_type=f32))

    @pl.when(i == 0)
    def _():
        pooled_ref[...] = part

    @pl.when(i > 0)
    def _():
        pooled_ref[...] += part

    @pl.when(i == pl.num_programs(0) - 1)
    def _():
        pooled = pooled_ref[...]
        hid = jnp.tanh(
            jnp.dot(pooled.astype(bf16), Wr1_ref[...].astype(bf16),
                    preferred_element_type=f32) + br1_ref[...])
        out_ref[...] = (jnp.dot(hid.astype(bf16), Wr2_ref[...].astype(bf16),
                                preferred_element_type=f32) + br2_ref[...])


def _row_spec(width):
    return pl.BlockSpec((BLK, width), lambda i: (i, 0))


_acc_spec = pl.BlockSpec((2, BLK, D), lambda i: (0, i, 0))
_dacc_spec = pl.BlockSpec((2, BLK, D), lambda i: (0, i, 0))
_w_spec = pl.BlockSpec((D, D), lambda i: (0, 0))
_vec_spec = pl.BlockSpec((1, D), lambda i: (0, 0))


def _tc0(dacc, x2, w2, bin2, Wg0):
    return pl.pallas_call(
        _tc0_body,
        grid=(GRID,),
        in_specs=[_dacc_spec, _row_spec(1), _vec_spec, _vec_spec, _w_spec],
        out_specs=[_row_spec(D), _row_spec(1)],
        out_shape=[jax.ShapeDtypeStruct((NP, D), f32),
                   jax.ShapeDtypeStruct((NP, 1), f32)],
    )(dacc, x2, w2, bin2, Wg0)


def _tc_mid(acc, gprev, dis, b2, W):
    return pl.pallas_call(
        _tc_mid_body,
        grid=(GRID,),
        in_specs=[_acc_spec, _row_spec(D), _row_spec(1), _vec_spec, _w_spec],
        out_specs=_row_spec(D),
        out_shape=jax.ShapeDtypeStruct((NP, D), f32),
    )(acc, gprev, dis, b2, W)


def _tc_fin(acc, gprev, dis, bg2_2, batch2, Wr1, br1_2, Wr2, br2_2):
    return pl.pallas_call(
        _tc_fin_body,
        grid=(GRID,),
        in_specs=[_acc_spec, _row_spec(D), _row_spec(1), _vec_spec,
                  _row_spec(1), _w_spec, _vec_spec,
                  pl.BlockSpec((D, 1), lambda i: (0, 0)),
                  pl.BlockSpec((1, 1), lambda i: (0, 0))],
        out_specs=pl.BlockSpec((G, 1), lambda i: (0, 0)),
        out_shape=jax.ShapeDtypeStruct((G, 1), f32),
        scratch_shapes=[pltpu.VMEM((G, D), f32)],
    )(acc, gprev, dis, bg2_2, batch2, Wr1, br1_2, Wr2, br2_2)


def kernel(x, edge_index, batch, W_in, b_in, Wg0, bg0, Wg1, bg1, Wg2, bg2,
           Wr1, br1, Wr2, br2):
    edges4 = jnp.transpose(edge_index.reshape(2, NTILES, NCH, CH),
                           (1, 0, 2, 3))
    zeros = jnp.zeros((NP, D), f32)
    ones = jnp.ones((NP, D), f32)
    x2 = jnp.pad(x, (0, NP - N)).reshape(NP, 1)
    batch2 = jnp.pad(batch, (0, NP - N), constant_values=G).reshape(NP, 1)

    iota_src = jnp.broadcast_to(jnp.arange(CH, dtype=jnp.int32),
                                (NTILES, NCH, CH))
    edges4d = edges4.at[:, 0].set(iota_src)
    dacc = _sc_mp(ones, edges4d, zeros)
    g0, dis = _tc0(dacc, x2, W_in, b_in.reshape(1, D), Wg0)
    a0 = _sc_mp(g0, edges4, zeros)
    g1 = _tc_mid(a0, g0, dis, bg0.reshape(1, D), Wg1)
    a1 = _sc_mp(g1, edges4, zeros)
    g2 = _tc_mid(a1, g1, dis, bg1.reshape(1, D), Wg2)
    a2 = _sc_mp(g2, edges4, zeros)
    return _tc_fin(a2, g2, dis, bg2.reshape(1, D), batch2,
                   Wr1, br1.reshape(1, D), Wr2, br2.reshape(1, 1))


# speedup vs baseline: 1.0697x; 1.0697x over previous
import jax
import jax.numpy as jnp
from jax import lax
from jax.experimental import pallas as pl
from jax.experimental.pallas import tpu as pltpu
from jax.experimental.pallas import tpu_sc as plsc

N = 10000
E = 320000
D = 128
G = 32

NP = 10240
NTILES = 32
EPT = E // NTILES
CH = 125
NCH = EPT // CH
ROWS_PT = NP // 16

BLK = 2048
GRID = NP // BLK

f32 = jnp.float32
bf16 = jnp.bfloat16

_mesh = plsc.VectorSubcoreMesh(core_axis_name="c", subcore_axis_name="s")


def _sc_mp_body(g_hbm, edges_hbm, zero_hbm, acc_hbm,
                edge_buf, buf0, sem0, acc_sh):
    c = lax.axis_index("c")
    s = lax.axis_index("s")
    wid = c * 16 + s
    pltpu.sync_copy(edges_hbm.at[wid], edge_buf)
    src_buf = edge_buf.at[0]
    dst_buf = edge_buf.at[1]
    rows = pl.ds(s * ROWS_PT, ROWS_PT)
    pltpu.sync_copy(zero_hbm.at[rows], acc_sh.at[rows])
    plsc.subcore_barrier()

    def body(j, carry):
        pltpu.async_copy(g_hbm.at[src_buf.at[j]], buf0, sem0).wait()
        pltpu.sync_copy(buf0, acc_sh.at[dst_buf.at[j]], add=True)
        return carry

    lax.fori_loop(0, NCH, body, 0)
    plsc.subcore_barrier()
    pltpu.sync_copy(acc_sh.at[rows], acc_hbm.at[c].at[rows])


_sc_mp = pl.kernel(
    _sc_mp_body,
    out_type=jax.ShapeDtypeStruct((2, NP, D), f32),
    mesh=_mesh,
    scratch_types=[
        pltpu.VMEM((2, NCH, CH), jnp.int32),
        pltpu.VMEM((CH, D), f32),
        pltpu.SemaphoreType.DMA,
        pltpu.VMEM_SHARED((NP, D), f32),
    ],
)

def _tc0_body(dacc_ref, x_ref, w_ref, bin_ref, W_ref, g_ref, dis_ref):
    deg = dacc_ref[0][:, 0:1] + dacc_ref[1][:, 0:1] + 1.0
    dis = 1.0 / jnp.sqrt(deg)
    h0 = x_ref[...] * w_ref[...] + bin_ref[...]
    t = jnp.dot(h0.astype(bf16), W_ref[...].astype(bf16),
                preferred_element_type=f32)
    g_ref[...] = dis * t
    dis_ref[...] = dis


def _tc_mid_body(acc_ref, gprev_ref, dis_ref, b_ref, W_ref, g_ref):
    dis = dis_ref[...]
    h = dis * (acc_ref[0] + acc_ref[1] + gprev_ref[...]) + b_ref[...]
    t = jnp.dot(h.astype(bf16), W_ref[...].astype(bf16),
                preferred_element_type=f32)
    g_ref[...] = dis * t


def _tc_fin_body(acc_ref, gprev_ref, dis_ref, bg2_ref, batch_ref,
                 Wr1_ref, br1_ref, Wr2_ref, br2_ref, out_ref, pooled_ref):
    i = pl.program_id(0)
    h3 = (dis_ref[...] * (acc_ref[0] + acc_ref[1] + gprev_ref[...])
          + bg2_ref[...])
    bb = batch_ref[...][:, 0]
    gids = lax.broadcasted_iota(jnp.int32, (G, BLK), 0)
    Bt = (bb[None, :] == gids).astype(bf16)
    hi = h3.astype(bf16)
    r = h3 - hi.astype(f32)
    mid = r.astype(bf16)
    lo = (r - mid.astype(f32)).astype(bf16)
    part = (jnp.dot(Bt, hi, preferred_element_type=f32)
            + jnp.dot(Bt, mid, preferred_element_type=f32)
            + jnp.dot(Bt, lo, preferred_element_type=f32))

    @pl.when(i == 0)
    def _():
        pooled_ref[...] = part

    @pl.when(i > 0)
    def _():
        pooled_ref[...] += part

    @pl.when(i == pl.num_programs(0) - 1)
    def _():
        pooled = pooled_ref[...]
        hid = jnp.tanh(
            jnp.dot(pooled.astype(bf16), Wr1_ref[...].astype(bf16),
                    preferred_element_type=f32) + br1_ref[...])
        out_ref[...] = (jnp.dot(hid.astype(bf16), Wr2_ref[...].astype(bf16),
                                preferred_element_type=f32) + br2_ref[...])


def _row_spec(width):
    return pl.BlockSpec((BLK, width), lambda i: (i, 0))


_acc_spec = pl.BlockSpec((2, BLK, D), lambda i: (0, i, 0))
_dacc_spec = pl.BlockSpec((2, BLK, D), lambda i: (0, i, 0))
_w_spec = pl.BlockSpec((D, D), lambda i: (0, 0))
_vec_spec = pl.BlockSpec((1, D), lambda i: (0, 0))


def _tc0(dacc, x2, w2, bin2, Wg0):
    return pl.pallas_call(
        _tc0_body,
        grid=(GRID,),
        in_specs=[_dacc_spec, _row_spec(1), _vec_spec, _vec_spec, _w_spec],
        out_specs=[_row_spec(D), _row_spec(1)],
        out_shape=[jax.ShapeDtypeStruct((NP, D), f32),
                   jax.ShapeDtypeStruct((NP, 1), f32)],
    )(dacc, x2, w2, bin2, Wg0)


def _tc_mid(acc, gprev, dis, b2, W):
    return pl.pallas_call(
        _tc_mid_body,
        grid=(GRID,),
        in_specs=[_acc_spec, _row_spec(D), _row_spec(1), _vec_spec, _w_spec],
        out_specs=_row_spec(D),
        out_shape=jax.ShapeDtypeStruct((NP, D), f32),
    )(acc, gprev, dis, b2, W)


def _tc_fin(acc, gprev, dis, bg2_2, batch2, Wr1, br1_2, Wr2, br2_2):
    return pl.pallas_call(
        _tc_fin_body,
        grid=(GRID,),
        in_specs=[_acc_spec, _row_spec(D), _row_spec(1), _vec_spec,
                  _row_spec(1), _w_spec, _vec_spec,
                  pl.BlockSpec((D, 1), lambda i: (0, 0)),
                  pl.BlockSpec((1, 1), lambda i: (0, 0))],
        out_specs=pl.BlockSpec((G, 1), lambda i: (0, 0)),
        out_shape=jax.ShapeDtypeStruct((G, 1), f32),
        scratch_shapes=[pltpu.VMEM((G, D), f32)],
    )(acc, gprev, dis, bg2_2, batch2, Wr1, br1_2, Wr2, br2_2)


def kernel(x, edge_index, batch, W_in, b_in, Wg0, bg0, Wg1, bg1, Wg2, bg2,
           Wr1, br1, Wr2, br2):
    edges4 = jnp.transpose(edge_index.reshape(2, NTILES, NCH, CH),
                           (1, 0, 2, 3))
    zeros = jnp.zeros((NP, D), f32)
    ones = jnp.ones((NP, D), f32)
    x2 = jnp.pad(x, (0, NP - N)).reshape(NP, 1)
    batch2 = jnp.pad(batch, (0, NP - N), constant_values=G).reshape(NP, 1)

    dacc = _sc_mp(ones, edges4, zeros)
    g0, dis = _tc0(dacc, x2, W_in, b_in.reshape(1, D), Wg0)
    a0 = _sc_mp(g0, edges4, zeros)
    g1 = _tc_mid(a0, g0, dis, bg0.reshape(1, D), Wg1)
    a1 = _sc_mp(g1, edges4, zeros)
    g2 = _tc_mid(a1, g1, dis, bg1.reshape(1, D), Wg2)
    a2 = _sc_mp(g2, edges4, zeros)
    return _tc_fin(a2, g2, dis, bg2.reshape(1, D), batch2,
                   Wr1, br1.reshape(1, D), Wr2, br2.reshape(1, 1))
